# baseline (speedup 1.0000x reference)
import functools

import jax
import jax.numpy as jnp
from jax import lax
from jax.experimental import pallas as pl
from jax.experimental.pallas import tpu as pltpu
from jax.experimental.pallas import tpu_sc as plsc

_LANES = 16


def _sc_body(row_base, rows_per_w, chunk_rows, D, B, n_chunks,
             x_hbm, pos_hbm, out_hbm,
             pos_v0, pos_v1,
             x_v0, x_v1, x_v2, x_v3, x_v4, x_v5, x_v6, x_v7,
             o_v0, o_v1, o_v2, o_v3,
             sp0, sp1,
             sx0, sx1, sx2, sx3, sx4, sx5, sx6, sx7,
             so0, so1, so2, so3):
    pos_v = (pos_v0, pos_v1)
    x_v = ((x_v0, x_v1), (x_v2, x_v3), (x_v4, x_v5), (x_v6, x_v7))
    o_v = (o_v0, o_v1, o_v2, o_v3)
    sp = (sp0, sp1)
    sx = ((sx0, sx1), (sx2, sx3), (sx4, sx5), (sx6, sx7))
    so = (so0, so1, so2, so3)

    wid = lax.axis_index("s") * 2 + lax.axis_index("c")
    row0 = wid * rows_per_w
    in_row0 = row_base + row0
    vec_iters = (chunk_rows * D) // _LANES
    row_iters = D // _LANES
    rsh = row_iters.bit_length() - 1
    jmask = row_iters - 1

    def nxt(c):
        return jnp.where(c + 1 == n_chunks, 0, c + 1)

    def start_pos(c, p):
        pltpu.make_async_copy(
            pos_hbm.at[pl.ds(in_row0 + c * chunk_rows, chunk_rows), :],
            pos_v[p], sp[p]).start()

    def wait_pos(p):
        pltpu.make_async_copy(
            pos_hbm.at[pl.ds(0, chunk_rows), :], pos_v[p], sp[p]).wait()

    def start_x(c, b, p):
        pltpu.make_async_copy(
            x_hbm.at[b, pl.ds(in_row0 + c * chunk_rows, chunk_rows), :],
            x_v[b][p], sx[b][p]).start()

    def wait_x(b, p):
        pltpu.make_async_copy(
            x_hbm.at[0, pl.ds(0, chunk_rows), :], x_v[b][p], sx[b][p]).wait()

    def start_out(c, b):
        pltpu.make_async_copy(
            o_v[b], out_hbm.at[b, pl.ds(row0 + c * chunk_rows, chunk_rows), :],
            so[b]).start()

    def wait_out(b):
        pltpu.make_async_copy(
            o_v[b], out_hbm.at[0, pl.ds(0, chunk_rows), :], so[b]).wait()

    def item(c, b, par, first):
        wait_x(b, par)
        start_x(nxt(c), b, 1 - par)
        if not first:
            wait_out(b)

        ob = o_v[b]
        xv = x_v[b][par]
        pv = pos_v[par]

        def add_loop(i, _):
            r = i >> rsh
            sl = pl.ds((i & jmask) * _LANES, _LANES)
            ob[r, sl] = xv[r, sl] + pv[r, sl]
            return ()

        lax.fori_loop(0, vec_iters, add_loop, (), unroll=8)
        start_out(c, b)

    def do_chunk(c, par, first_chunk):
        wait_pos(par)
        start_pos(nxt(c), 1 - par)
        for b in range(B):
            item(c, b, par, first=first_chunk)

    start_pos(0, 0)
    for b in range(B):
        start_x(0, b, 0)
    do_chunk(0, 0, True)
    do_chunk(1, 1, False)

    def pair_body(c2, _):
        do_chunk(2 * c2, 0, False)
        do_chunk(2 * c2 + 1, 1, False)
        return ()

    lax.fori_loop(1, n_chunks // 2, pair_body, ())

    wait_pos(0)
    for b in range(B):
        wait_x(b, 0)
        wait_out(b)


def _sc_add(x, pos, B, D, row_base, sc_rows):
    info = plsc.get_sparse_core_info()
    nw = info.num_cores * info.num_subcores
    rows_per_w = sc_rows // nw
    chunk_rows = next(
        cr for cr in (8, 16)
        if rows_per_w % cr == 0
        and (rows_per_w // cr) >= 2
        and (rows_per_w // cr) % 2 == 0)
    n_chunks = rows_per_w // chunk_rows
    mesh = plsc.VectorSubcoreMesh(core_axis_name="c", subcore_axis_name="s")
    f = pl.kernel(
        functools.partial(_sc_body, row_base, rows_per_w, chunk_rows, D, B,
                          n_chunks),
        mesh=mesh,
        out_type=jax.ShapeDtypeStruct((B, sc_rows, D), jnp.float32),
        scratch_types=(
            [pltpu.VMEM((chunk_rows, D), jnp.float32)] * 14
            + [pltpu.SemaphoreType.DMA] * 14
        ),
    )
    return f(x, pos)


def kernel(x, pos_table):
    B, S, D = x.shape
    assert S <= pos_table.shape[0] and S % 32 == 0
    return _sc_add(x, pos_table[:S], B, D, 0, S)


# speedup vs baseline: 1.2456x; 1.2456x over previous
import functools

import jax
import jax.numpy as jnp
from jax import lax
from jax.experimental import pallas as pl
from jax.experimental.pallas import tpu as pltpu
from jax.experimental.pallas import tpu_sc as plsc

_LANES = 16
_NSLOT = 3


def _sc_body(rows_per_w, chunk_rows, D, B, n_chunks,
             x_hbm, pos_hbm, out_hbm,
             x_v0, x_v1, x_v2, p_v0, p_v1, p_v2,
             sx0, sx1, sx2, sp0, sp1, sp2, so0, so1, so2):
    x_v = (x_v0, x_v1, x_v2)
    p_v = (p_v0, p_v1, p_v2)
    sx = (sx0, sx1, sx2)
    sp = (sp0, sp1, sp2)
    so = (so0, so1, so2)

    wid = lax.axis_index("s") * 2 + lax.axis_index("c")
    row0 = wid * rows_per_w
    row_iters = D // _LANES
    rsh = row_iters.bit_length() - 1
    jmask = row_iters - 1
    vec_iters = chunk_rows * row_iters

    def nxt(c):
        return jnp.where(c + 1 == n_chunks, 0, c + 1)

    def start_in(c, s):
        rows = pl.ds(row0 + c * chunk_rows, chunk_rows)
        pltpu.make_async_copy(x_hbm.at[:, rows, :], x_v[s], sx[s]).start()
        pltpu.make_async_copy(pos_hbm.at[rows, :], p_v[s], sp[s]).start()

    def wait_in(s):
        rows0 = pl.ds(0, chunk_rows)
        pltpu.make_async_copy(x_hbm.at[:, rows0, :], x_v[s], sx[s]).wait()
        pltpu.make_async_copy(pos_hbm.at[rows0, :], p_v[s], sp[s]).wait()

    def start_out(c, s):
        rows = pl.ds(row0 + c * chunk_rows, chunk_rows)
        pltpu.make_async_copy(x_v[s], out_hbm.at[:, rows, :], so[s]).start()

    def wait_out(s):
        rows0 = pl.ds(0, chunk_rows)
        pltpu.make_async_copy(x_v[s], out_hbm.at[:, rows0, :], so[s]).wait()

    def chunk_body(c, s, first):
        sn = (s + 1) % _NSLOT
        if not first:
            wait_out(sn)
        start_in(nxt(c), sn)
        wait_in(s)

        xv = x_v[s]
        pv = p_v[s]

        def add_loop(i, _):
            r = i >> rsh
            sl = pl.ds((i & jmask) * _LANES, _LANES)
            vp = pv[r, sl]
            for b in range(B):
                xv[b, r, sl] = xv[b, r, sl] + vp
            return ()

        lax.fori_loop(0, vec_iters, add_loop, (), unroll=4)
        start_out(c, s)

    start_in(0, 0)
    chunk_body(0, 0, True)
    chunk_body(1, 1, True)

    branches = [functools.partial(chunk_body, s=s, first=False)
                for s in range(_NSLOT)]

    def loop_body(c, _):
        lax.switch(c % _NSLOT, branches, c)
        return ()

    lax.fori_loop(2, n_chunks, loop_body, ())

    last = (n_chunks - 1) % _NSLOT
    wait_in((last + 1) % _NSLOT)
    wait_out((last + _NSLOT - 1) % _NSLOT)
    wait_out(last)


def _sc_add(x, pos, B, S, D):
    info = plsc.get_sparse_core_info()
    nw = info.num_cores * info.num_subcores
    rows_per_w = S // nw
    chunk_rows = 8
    n_chunks = rows_per_w // chunk_rows
    assert rows_per_w % chunk_rows == 0 and n_chunks >= 3
    mesh = plsc.VectorSubcoreMesh(core_axis_name="c", subcore_axis_name="s")
    f = pl.kernel(
        functools.partial(_sc_body, rows_per_w, chunk_rows, D, B, n_chunks),
        mesh=mesh,
        out_type=jax.ShapeDtypeStruct((B, S, D), jnp.float32),
        scratch_types=(
            [pltpu.VMEM((B, chunk_rows, D), jnp.float32)] * _NSLOT
            + [pltpu.VMEM((chunk_rows, D), jnp.float32)] * _NSLOT
            + [pltpu.SemaphoreType.DMA] * (3 * _NSLOT)
        ),
    )
    return f(x, pos)


def kernel(x, pos_table):
    B, S, D = x.shape
    assert S <= pos_table.shape[0] and S % 32 == 0
    return _sc_add(x, pos_table[:S], B, S, D)
